# baseline (speedup 1.0000x reference)
import functools

import jax
import jax.numpy as jnp
from jax import lax
from jax.experimental import pallas as pl
from jax.experimental.pallas import tpu as pltpu
from jax.experimental.pallas import tpu_sc as plsc

DIM = 64
GAMMA = 12.0


@functools.cache
def _make_sc_kernel(B: int, n_rows: int):
    info = plsc.get_sparse_core_info()
    NC, NS, L = info.num_cores, info.num_subcores, info.num_lanes
    NW = NC * NS
    BW = B // NW
    NG = BW // L
    mesh = plsc.VectorSubcoreMesh(core_axis_name="c", subcore_axis_name="s")

    @functools.partial(
        pl.kernel,
        mesh=mesh,
        compiler_params=pltpu.CompilerParams(needs_layout_passes=False),
        out_type=jax.ShapeDtypeStruct((B,), jnp.float32),
        scratch_types=[
            pltpu.VMEM((BW,), jnp.int32),
            pltpu.VMEM((BW,), jnp.int32),
            pltpu.VMEM((BW,), jnp.int32),
            pltpu.VMEM((L, 2 * DIM), jnp.float32),
            pltpu.VMEM((L, 2 * DIM), jnp.float32),
            pltpu.VMEM((L, 2 * DIM), jnp.float32),
            pltpu.VMEM((BW,), jnp.float32),
            pltpu.SemaphoreType.DMA,
        ],
    )
    def k(hidx_hbm, ridx_hbm, tidx_hbm, ent_hbm, rel_hbm, out_hbm,
          hidx_v, ridx_v, tidx_v, h_v, r_v, t_v, out_v, sem):
        wid = lax.axis_index("s") * NC + lax.axis_index("c")
        base = wid * BW
        pltpu.sync_copy(hidx_hbm.at[pl.ds(base, BW)], hidx_v)
        pltpu.sync_copy(ridx_hbm.at[pl.ds(base, BW)], ridx_v)
        pltpu.sync_copy(tidx_hbm.at[pl.ds(base, BW)], tidx_v)

        lanes = lax.iota(jnp.int32, L)

        def group(g, carry):
            sl = pl.ds(g * L, L)
            his = hidx_v[sl]
            ris = ridx_v[sl]
            tis = tidx_v[sl]
            cps = [
                pltpu.async_copy(ent_hbm.at[his >> 1], h_v, sem),
                pltpu.async_copy(rel_hbm.at[ris >> 1], r_v, sem),
                pltpu.async_copy(ent_hbm.at[tis >> 1], t_v, sem),
            ]
            for c in cps:
                c.wait()

            hoff = (his & 1) * DIM
            roff = (ris & 1) * DIM
            toff = (tis & 1) * DIM
            acc = jnp.zeros((L,), jnp.float32)
            for d in range(DIM):
                h = plsc.load_gather(h_v, [lanes, hoff + d])
                r = plsc.load_gather(r_v, [lanes, roff + d])
                t = plsc.load_gather(t_v, [lanes, toff + d])
                acc = acc + jnp.abs(h + r - t)
            out_v[sl] = GAMMA - acc
            return carry

        lax.fori_loop(0, NG, group, 0)
        pltpu.sync_copy(out_v, out_hbm.at[pl.ds(base, BW)])

    return k


@jax.jit
def kernel(sample, entity_embedding, relation_embedding):
    B = sample.shape[0]
    hidx = sample[:, 0]
    ridx = sample[:, 1]
    tidx = sample[:, 2]
    n, d = entity_embedding.shape
    ent2 = entity_embedding.reshape(n // 2, 2 * d)
    rel2 = relation_embedding.reshape(n // 2, 2 * d)
    score = _make_sc_kernel(B, n // 2)(hidx, ridx, tidx, ent2, rel2)
    return score.reshape(B, 1)
